# baseline (speedup 1.0000x reference)
import functools

import jax
import jax.numpy as jnp
from jax import lax
from jax.experimental import pallas as pl
from jax.experimental.pallas import tpu as pltpu
from jax.experimental.pallas import tpu_sc as plsc

D = 768
E = 8
NSEG = D // 16
NC = 2
NS = 16
NW = NC * NS
CH = 128
B = 4


def _lane_permute(v, perm):
    dnums = lax.GatherDimensionNumbers(
        offset_dims=(), collapsed_slice_dims=(0,), start_index_map=(0,))
    return lax.gather(v, perm[:, None], dnums, (1,),
                      mode=lax.GatherScatterMode.PROMISE_IN_BOUNDS)


def _lane_sum(v, iota):
    for k in (8, 4, 2, 1):
        v = v + _lane_permute(v, iota ^ k)
    return v


def _sc_body(x_hbm, w_hbm, o_hbm, xbuf, wbuf, obuf, sem):
    c = lax.axis_index("c")
    s = lax.axis_index("s")
    wid = s * NC + c
    tk = x_hbm.shape[0] // NW
    base = wid * tk
    iota = lax.iota(jnp.int32, 16)

    pltpu.sync_copy(w_hbm, wbuf)

    def chunk_body(ci, _):
        rb = base + ci * CH
        pltpu.sync_copy(x_hbm.at[pl.ds(rb, CH)], xbuf)

        def tb_body(bi, _):
            t0 = bi * B
            accs = [[jnp.zeros((16,), jnp.float32) for _ in range(E)]
                    for _ in range(B)]
            for j in range(NSEG):
                wv = [wbuf[e, pl.ds(j * 16, 16)] for e in range(E)]
                for t in range(B):
                    xv = xbuf[t0 + t, pl.ds(j * 16, 16)]
                    for e in range(E):
                        accs[t][e] = accs[t][e] + xv * wv[e]
            for t in range(0, B, 2):
                out16 = jnp.zeros((16,), jnp.float32)
                for dt in (0, 1):
                    for e in range(E):
                        r = _lane_sum(accs[t + dt][e], iota)
                        out16 = jnp.where(iota == (e + 8 * dt), r, out16)
                off = pl.multiple_of((t0 + t) * E, 16)
                obuf[pl.ds(off, 16)] = out16
            return 0

        lax.fori_loop(0, CH // B, tb_body, 0)
        pltpu.sync_copy(obuf, o_hbm.at[pl.ds(pl.multiple_of(rb * E, 8), CH * E)])
        return 0

    lax.fori_loop(0, tk // CH, chunk_body, 0)


def kernel(x, W):
    T = x.shape[0]
    mesh = plsc.VectorSubcoreMesh(core_axis_name="c", subcore_axis_name="s")
    k = functools.partial(
        pl.kernel,
        out_type=jax.ShapeDtypeStruct((T * E,), jnp.float32),
        mesh=mesh,
        scratch_types=[
            pltpu.VMEM((CH, D), jnp.float32),
            pltpu.VMEM((E, D), jnp.float32),
            pltpu.VMEM((CH * E,), jnp.float32),
            pltpu.SemaphoreType.DMA,
        ],
    )(_sc_body)
    out = k(x, W)
    return out.reshape(T, E)


# speedup vs baseline: 1.0330x; 1.0330x over previous
import functools

import jax
import jax.numpy as jnp
from jax import lax
from jax.experimental import pallas as pl
from jax.experimental.pallas import tpu as pltpu
from jax.experimental.pallas import tpu_sc as plsc

D = 768
E = 8
NSEG = D // 16
NC = 2
NS = 16
NW = NC * NS
CH = 128
B = 4


def _lane_permute(v, perm):
    dnums = lax.GatherDimensionNumbers(
        offset_dims=(), collapsed_slice_dims=(0,), start_index_map=(0,))
    return lax.gather(v, perm[:, None], dnums, (1,),
                      mode=lax.GatherScatterMode.PROMISE_IN_BOUNDS)


def _tree_reduce16(vs, iota):
    k = 1
    while len(vs) > 1:
        bit = (iota & k) != 0
        perm = iota ^ k
        nxt = []
        for i in range(0, len(vs), 2):
            a, b = vs[i], vs[i + 1]
            sel = jnp.where(bit, b, a)
            other = jnp.where(bit, a, b)
            nxt.append(sel + _lane_permute(other, perm))
        vs = nxt
        k *= 2
    return vs[0]


def _sc_body(x_hbm, w_hbm, o_hbm, xbuf, wbuf, obuf, sem):
    c = lax.axis_index("c")
    s = lax.axis_index("s")
    wid = s * NC + c
    tk = x_hbm.shape[0] // NW
    base = wid * tk
    iota = lax.iota(jnp.int32, 16)

    pltpu.sync_copy(w_hbm, wbuf)

    def chunk_body(ci, _):
        rb = base + ci * CH
        pltpu.sync_copy(x_hbm.at[pl.ds(rb, CH)], xbuf)

        @plsc.parallel_loop(0, CH // B)
        def tb_body(bi):
            t0 = bi * B
            accs = [[jnp.zeros((16,), jnp.float32) for _ in range(E)]
                    for _ in range(B)]
            for j in range(NSEG):
                wv = [wbuf[e, pl.ds(j * 16, 16)] for e in range(E)]
                for t in range(B):
                    xv = xbuf[t0 + t, pl.ds(j * 16, 16)]
                    for e in range(E):
                        accs[t][e] = accs[t][e] + xv * wv[e]
            for t in range(0, B, 2):
                out16 = _tree_reduce16(accs[t] + accs[t + 1], iota)
                off = pl.multiple_of((t0 + t) * E, 16)
                obuf[pl.ds(off, 16)] = out16

        pltpu.sync_copy(obuf, o_hbm.at[pl.ds(pl.multiple_of(rb * E, 8), CH * E)])
        return 0

    lax.fori_loop(0, tk // CH, chunk_body, 0)


def kernel(x, W):
    T = x.shape[0]
    mesh = plsc.VectorSubcoreMesh(core_axis_name="c", subcore_axis_name="s")
    k = functools.partial(
        pl.kernel,
        out_type=jax.ShapeDtypeStruct((T * E,), jnp.float32),
        mesh=mesh,
        scratch_types=[
            pltpu.VMEM((CH, D), jnp.float32),
            pltpu.VMEM((E, D), jnp.float32),
            pltpu.VMEM((CH * E,), jnp.float32),
            pltpu.SemaphoreType.DMA,
        ],
    )(_sc_body)
    out = k(x, W)
    return out.reshape(T, E)


# speedup vs baseline: 5.0388x; 4.8777x over previous
import functools

import jax
import jax.numpy as jnp
from jax import lax
from jax.experimental import pallas as pl
from jax.experimental.pallas import tpu as pltpu

D = 768
E = 8
NBUF = 4
BLK = 1024


def _tc_body(x_hbm, wt_ref, o_ref, bufs, sems):
    T = x_hbm.shape[0]
    nblk = T // BLK
    wt = wt_ref[...]

    def start(i, p):
        pltpu.make_async_copy(
            x_hbm.at[pl.ds(i * BLK, BLK)], bufs.at[p], sems.at[p]).start()

    for b in range(NBUF):
        start(b, b)

    def step(i, _):
        p = lax.rem(i, NBUF)
        pltpu.make_async_copy(
            x_hbm.at[pl.ds(i * BLK, BLK)], bufs.at[p], sems.at[p]).wait()
        o_ref[pl.ds(i * BLK, BLK)] = jnp.dot(
            bufs[p], wt, preferred_element_type=jnp.float32)

        @pl.when(i + NBUF < nblk)
        def _():
            start(i + NBUF, p)

        return 0

    lax.fori_loop(0, nblk, step, 0)


def kernel(x, W):
    T = x.shape[0]
    Wt = W.T
    out = pl.pallas_call(
        _tc_body,
        in_specs=[
            pl.BlockSpec(memory_space=pl.ANY),
            pl.BlockSpec(memory_space=pltpu.VMEM),
        ],
        out_specs=pl.BlockSpec(memory_space=pltpu.VMEM),
        out_shape=jax.ShapeDtypeStruct((T, E), jnp.float32),
        scratch_shapes=[
            pltpu.VMEM((NBUF, BLK, D), jnp.float32),
            pltpu.SemaphoreType.DMA((NBUF,)),
        ],
    )(x, Wt)
    return out
